# baseline (speedup 1.0000x reference)
import functools

import jax
import jax.numpy as jnp
from jax import lax
from jax.experimental import pallas as pl
from jax.experimental.pallas import tpu as pltpu
from jax.experimental.pallas import tpu_sc as plsc


def _sc_gather(table, idx_flat):
    info = plsc.get_sparse_core_info()
    nw = info.num_cores * info.num_subcores
    n = idx_flat.shape[0]
    d = table.shape[1]
    n_per_w = n // nw
    mesh = plsc.VectorSubcoreMesh(core_axis_name="c", subcore_axis_name="s")

    @functools.partial(
        pl.kernel,
        mesh=mesh,
        out_type=jax.ShapeDtypeStruct((n, d), jnp.float32),
        compiler_params=pltpu.CompilerParams(use_tc_tiling_on_sc=False),
        scratch_types=[
            pltpu.VMEM((n_per_w,), jnp.int32),
            pltpu.VMEM((n_per_w, d), jnp.float32),
            pltpu.SemaphoreType.DMA,
        ],
    )
    def k(table_hbm, idx_hbm, out_hbm, idx_v, rows_v, sem):
        wid = lax.axis_index("s") * info.num_cores + lax.axis_index("c")
        base = wid * n_per_w
        pltpu.sync_copy(idx_hbm.at[pl.ds(base, n_per_w)], idx_v)
        pltpu.async_copy(table_hbm.at[idx_v], rows_v, sem).wait()
        pltpu.sync_copy(rows_v, out_hbm.at[pl.ds(base, n_per_w)])

    return k(table, idx_flat)


_BN = 1024
_NBUF = 6
_CH = 128
_NCH = _BN // _CH


def _mm_body(nsteps, vstart_last, w_hbm, flat_ref, bias_ref, out_hbm,
             extra_hbm, w_ring, out_ring, w_sem, out_sem, extra_sem):
    j = pl.program_id(0)
    last = nsteps - 1

    def w_chunk_copy(block, slot, c):
        start = pl.multiple_of(jnp.minimum(block * _BN, vstart_last), 32)
        return pltpu.make_async_copy(
            w_hbm.at[pl.ds(start + c * _CH, _CH)],
            w_ring.at[slot, pl.ds(c * _CH, _CH)],
            w_sem.at[slot],
        )

    def issue_block(block):
        slot = lax.rem(block, _NBUF)
        for c in range(_NCH):
            w_chunk_copy(block, slot, c).start()

    def out_copy(block):
        return pltpu.make_async_copy(
            out_ring.at[lax.rem(block, 2)],
            out_hbm.at[:, pl.ds(pl.multiple_of(block * _BN, _BN), _BN)],
            out_sem.at[lax.rem(block, 2)],
        )

    def extra_copy():
        return pltpu.make_async_copy(
            out_ring.at[last % 2],
            extra_hbm,
            extra_sem,
        )

    @pl.when(j == 0)
    def _():
        for b in range(_NBUF - 1):
            issue_block(b)

    slot = lax.rem(j, _NBUF)
    for c in range(_NCH):
        w_chunk_copy(j, slot, c).wait()

    @pl.when(j >= 2)
    def _():
        out_copy(j - 2).wait()

    @pl.when(j + _NBUF - 1 <= last)
    def _():
        issue_block(j + _NBUF - 1)

    acc = lax.dot_general(
        flat_ref[...], w_ring[slot],
        (((1,), (1,)), ((), ())),
        preferred_element_type=jnp.float32,
    )
    acc = acc + bias_ref[pl.ds(j, 1), :]
    out_ring[lax.rem(j, 2)] = acc

    @pl.when(j < last)
    def _():
        out_copy(j).start()

    @pl.when(j == last)
    def _():
        extra_copy().start()
        out_copy(last - 1).wait()
        extra_copy().wait()


def _tc_matmul(flat, linear_w, linear_b):
    b, h = flat.shape
    v = linear_w.shape[0]
    nsteps = pl.cdiv(v, _BN)
    vstart_last = v - _BN
    starts = jnp.minimum(jnp.arange(nsteps) * _BN, vstart_last)
    bias2d = linear_b[starts[:, None] + jnp.arange(_BN)[None, :]]
    out_main, out_extra = pl.pallas_call(
        functools.partial(_mm_body, nsteps, vstart_last),
        grid=(nsteps,),
        in_specs=[
            pl.BlockSpec(memory_space=pl.ANY),
            pl.BlockSpec((b, h), lambda j: (0, 0)),
            pl.BlockSpec((nsteps, _BN), lambda j: (0, 0)),
        ],
        out_specs=[
            pl.BlockSpec(memory_space=pl.ANY),
            pl.BlockSpec(memory_space=pl.ANY),
        ],
        out_shape=[
            jax.ShapeDtypeStruct((b, v), jnp.float32),
            jax.ShapeDtypeStruct((b, _BN), jnp.float32),
        ],
        scratch_shapes=[
            pltpu.VMEM((_NBUF, _BN, h), jnp.float32),
            pltpu.VMEM((2, b, _BN), jnp.float32),
            pltpu.SemaphoreType.DMA((_NBUF,)),
            pltpu.SemaphoreType.DMA((2,)),
            pltpu.SemaphoreType.DMA,
        ],
        compiler_params=pltpu.CompilerParams(
            dimension_semantics=("arbitrary",),
        ),
    )(linear_w, flat, bias2d)
    return lax.dynamic_update_slice(out_main, out_extra, (0, vstart_last))


def kernel(x, embedding_table, linear_w, linear_b):
    b, l = x.shape
    d = embedding_table.shape[1]
    flat = _sc_gather(embedding_table, x.reshape(-1)).reshape(b, l * d)
    return _tc_matmul(flat, linear_w, linear_b)
